# Initial kernel scaffold; baseline (speedup 1.0000x reference)
import jax
import jax.numpy as jnp
from jax.experimental import pallas as pl


def kernel(x, cheb_weight, cheb_bias):
    raise NotImplementedError("write your pallas kernel here")


# speedup vs baseline: 1.2598x; 1.2598x over previous
import jax
import jax.numpy as jnp
import numpy as np
from jax.experimental import pallas as pl
from jax.experimental.pallas import tpu as pltpu

_J = 17
_NP = 5
_C = 128
_K = 3

_PART_JOINTS = [[1, 2, 3], [4, 5, 6], [0, 7, 8, 9, 10], [11, 12, 13], [14, 15, 16]]
_JOINT_TO_PART = [2, 0, 0, 0, 1, 1, 1, 2, 2, 2, 2, 3, 3, 3, 4, 4, 4]


def _graph_constants():
    edges = np.array([[0, 2], [1, 2], [2, 3], [2, 4]], dtype=np.int64)
    A = np.zeros((_NP, _NP), dtype=np.float64)
    A[edges[:, 0], edges[:, 1]] = 1.0
    A = np.maximum(A, A.T)
    A = A + np.eye(_NP)
    A = A / A.sum(axis=1, keepdims=True)
    d = A.sum(axis=-1)
    D = np.diag(d ** -0.5)
    L = np.eye(_NP) - D @ A @ D
    return L.astype(np.float32)


_L = _graph_constants()


def _fused_body(x_ref, w_ref, b_ref, o_ref):
    xb = x_ref[...]

    pf = []
    for joints in _PART_JOINTS:
        acc = xb[:, joints[0] * _C:(joints[0] + 1) * _C]
        for j in joints[1:]:
            acc = acc + xb[:, j * _C:(j + 1) * _C]
        pf.append(acc * np.float32(1.0 / len(joints)))

    def lmix(rows):
        out = []
        for p in range(_NP):
            acc = None
            for q in range(_NP):
                c = float(_L[p, q])
                if c == 0.0:
                    continue
                term = rows[q] * np.float32(c)
                acc = term if acc is None else acc + term
            out.append(acc)
        return out

    y1 = lmix(pf)
    ly1 = lmix(y1)
    y2 = [np.float32(2.0) * ly1[p] - pf[p] for p in range(_NP)]

    bias = b_ref[...]
    w = w_ref[...]

    h = []
    for p in range(_NP):
        z = jnp.concatenate([pf[p], y1[p], y2[p]], axis=-1)
        hp = jax.lax.dot_general(
            z, w, (((1,), (0,)), ((), ())),
            preferred_element_type=jnp.float32)
        h.append(hp + bias)

    o_ref[...] = jnp.concatenate([h[p] for p in _JOINT_TO_PART], axis=-1)


def kernel(x, cheb_weight, cheb_bias):
    B = x.shape[0]
    bt = 512
    x2 = x.reshape(B, _J * _C)
    wstack = cheb_weight.reshape(_K * _C, _C)
    bias2 = cheb_bias.reshape(1, _C)

    out2 = pl.pallas_call(
        _fused_body,
        grid=(B // bt,),
        in_specs=[
            pl.BlockSpec((bt, _J * _C), lambda i: (i, 0)),
            pl.BlockSpec((_K * _C, _C), lambda i: (0, 0)),
            pl.BlockSpec((1, _C), lambda i: (0, 0)),
        ],
        out_specs=pl.BlockSpec((bt, _J * _C), lambda i: (i, 0)),
        out_shape=jax.ShapeDtypeStruct((B, _J * _C), x.dtype),
        compiler_params=pltpu.CompilerParams(
            dimension_semantics=("parallel",)),
    )(x2, wstack, bias2)
    return out2.reshape(B, _J, _C)


# speedup vs baseline: 1.8964x; 1.5053x over previous
import jax
import jax.numpy as jnp
import numpy as np
from jax.experimental import pallas as pl
from jax.experimental.pallas import tpu as pltpu

_J = 17
_NP = 5
_C = 128
_K = 3
_BT = 512

_PART_JOINTS = [[1, 2, 3], [4, 5, 6], [0, 7, 8, 9, 10], [11, 12, 13], [14, 15, 16]]
_JOINT_TO_PART = [2, 0, 0, 0, 1, 1, 1, 2, 2, 2, 2, 3, 3, 3, 4, 4, 4]


def _graph_constants():
    edges = np.array([[0, 2], [1, 2], [2, 3], [2, 4]], dtype=np.int64)
    A = np.zeros((_NP, _NP), dtype=np.float64)
    A[edges[:, 0], edges[:, 1]] = 1.0
    A = np.maximum(A, A.T)
    A = A + np.eye(_NP)
    A = A / A.sum(axis=1, keepdims=True)
    d = A.sum(axis=-1)
    D = np.diag(d ** -0.5)
    L = np.eye(_NP) - D @ A @ D
    return L.astype(np.float32)


_L = _graph_constants()


def _body(x_hbm, w_ref, b_ref, o_hbm, xs, hs, in_sems, out_sems):
    i = pl.program_id(0)
    nb = pl.num_programs(0)
    slot = jax.lax.rem(i, 2)

    def in_copy(block, s, j):
        return pltpu.make_async_copy(
            x_hbm.at[pl.ds(block * _BT, _BT), j], xs.at[s, j], in_sems.at[s])

    def out_copy(block, s, j):
        return pltpu.make_async_copy(
            hs.at[s, _JOINT_TO_PART[j]],
            o_hbm.at[pl.ds(block * _BT, _BT), j], out_sems.at[s])

    def start_in(block, s):
        for j in range(_J):
            in_copy(block, s, j).start()

    def wait_in(s):
        for j in range(_J):
            in_copy(0, s, j).wait()

    def start_out(block, s):
        for j in range(_J):
            out_copy(block, s, j).start()

    def wait_out(s):
        for j in range(_J):
            out_copy(0, s, j).wait()

    @pl.when(i == 0)
    def _():
        start_in(0, 0)

    @pl.when(i + 1 < nb)
    def _():
        start_in(i + 1, 1 - slot)

    wait_in(slot)

    pf = []
    for joints in _PART_JOINTS:
        acc = xs[slot, joints[0]]
        for j in joints[1:]:
            acc = acc + xs[slot, j]
        pf.append(acc * np.float32(1.0 / len(joints)))

    def lmix(rows):
        out = []
        for p in range(_NP):
            acc = None
            for q in range(_NP):
                c = float(_L[p, q])
                if c == 0.0:
                    continue
                term = rows[q] * np.float32(c)
                acc = term if acc is None else acc + term
            out.append(acc)
        return out

    y1 = lmix(pf)
    ly1 = lmix(y1)
    y2 = [np.float32(2.0) * ly1[p] - pf[p] for p in range(_NP)]

    bias = b_ref[...]
    w = w_ref[...]

    @pl.when(i >= 2)
    def _():
        wait_out(slot)

    for p in range(_NP):
        z = jnp.concatenate([pf[p], y1[p], y2[p]], axis=-1)
        hp = jax.lax.dot_general(
            z, w, (((1,), (0,)), ((), ())),
            preferred_element_type=jnp.float32)
        hs[slot, p] = hp + bias

    start_out(i, slot)

    @pl.when(i == nb - 1)
    def _():
        wait_out(1 - slot)
        wait_out(slot)


def kernel(x, cheb_weight, cheb_bias):
    B = x.shape[0]
    nb = B // _BT
    wstack = cheb_weight.reshape(_K * _C, _C)
    bias2 = cheb_bias.reshape(1, _C)

    return pl.pallas_call(
        _body,
        grid=(nb,),
        in_specs=[
            pl.BlockSpec(memory_space=pltpu.MemorySpace.HBM),
            pl.BlockSpec((_K * _C, _C), lambda i: (0, 0)),
            pl.BlockSpec((1, _C), lambda i: (0, 0)),
        ],
        out_specs=pl.BlockSpec(memory_space=pltpu.MemorySpace.HBM),
        out_shape=jax.ShapeDtypeStruct((B, _J, _C), x.dtype),
        scratch_shapes=[
            pltpu.VMEM((2, _J, _BT, _C), jnp.float32),
            pltpu.VMEM((2, _NP, _BT, _C), jnp.float32),
            pltpu.SemaphoreType.DMA((2,)),
            pltpu.SemaphoreType.DMA((2,)),
        ],
        compiler_params=pltpu.CompilerParams(
            dimension_semantics=("arbitrary",)),
    )(x, wstack, bias2)
